# baseline (speedup 1.0000x reference)
import functools

import jax
import jax.numpy as jnp
from jax import lax
from jax.experimental import pallas as pl
from jax.experimental.pallas import tpu as pltpu
from jax.experimental.pallas import tpu_sc as plsc

G = 128


def kernel(input_feature, table):
    B, N, L = input_feature.shape
    V, D = table.shape
    T = B * N * L

    info = plsc.get_sparse_core_info()
    NW = info.num_cores * info.num_subcores
    per_w = T // NW
    n_g = per_w // G
    assert per_w * NW == T and n_g * G == per_w

    idx = input_feature.reshape(NW, n_g, G).astype(jnp.int32)
    mesh = plsc.VectorSubcoreMesh(core_axis_name="c", subcore_axis_name="s")

    @functools.partial(
        pl.kernel,
        mesh=mesh,
        out_type=jax.ShapeDtypeStruct((NW, n_g, G, D), jnp.float32),
        scratch_types=[
            pltpu.VMEM((n_g, G), jnp.int32),
            pltpu.VMEM((G, D), jnp.float32),
            pltpu.VMEM((G, D), jnp.float32),
            pltpu.SemaphoreType.DMA,
            pltpu.SemaphoreType.DMA,
        ],
        compiler_params=pltpu.CompilerParams(use_tc_tiling_on_sc=False),
    )
    def emb(idx_hbm, table_hbm, out_hbm, idx_v, buf0, buf1, sem0, sem1):
        wid = lax.axis_index("s") * info.num_cores + lax.axis_index("c")
        pltpu.sync_copy(idx_hbm.at[wid], idx_v)
        bufs = (buf0, buf1)
        sems = (sem0, sem1)

        pltpu.async_copy(table_hbm.at[idx_v.at[0]], buf0, sem0)

        def body(o, carry):
            for b in range(2):
                g = o * 2 + b
                nxt = g + 1

                @pl.when(nxt < n_g)
                def _():
                    pltpu.async_copy(
                        table_hbm.at[idx_v.at[nxt]], bufs[1 - b], sems[1 - b]
                    )

                pltpu.make_async_copy(
                    table_hbm.at[idx_v.at[g]], bufs[b], sems[b]
                ).wait()
                pltpu.sync_copy(bufs[b], out_hbm.at[wid, g])
            return carry

        lax.fori_loop(0, n_g // 2, body, 0)

    out = emb(idx, table)
    return out.reshape(B, N, L, D)


# speedup vs baseline: 1.0006x; 1.0006x over previous
import functools

import jax
import jax.numpy as jnp
from jax import lax
from jax.experimental import pallas as pl
from jax.experimental.pallas import tpu as pltpu
from jax.experimental.pallas import tpu_sc as plsc

G = 128


def kernel(input_feature, table):
    B, N, L = input_feature.shape
    V, D = table.shape
    T = B * N * L

    info = plsc.get_sparse_core_info()
    NW = info.num_cores * info.num_subcores
    per_w = T // NW
    n_g = per_w // G
    assert per_w * NW == T and n_g * G == per_w

    idx = input_feature.reshape(-1).astype(jnp.int32)
    mesh = plsc.VectorSubcoreMesh(core_axis_name="c", subcore_axis_name="s")

    @functools.partial(
        pl.kernel,
        mesh=mesh,
        out_type=jax.ShapeDtypeStruct((T, D), jnp.float32),
        scratch_types=[
            pltpu.VMEM((per_w,), jnp.int32),
            pltpu.VMEM((G, D), jnp.float32),
            pltpu.VMEM((G, D), jnp.float32),
            pltpu.SemaphoreType.DMA,
            pltpu.SemaphoreType.DMA,
        ],
        compiler_params=pltpu.CompilerParams(use_tc_tiling_on_sc=False),
    )
    def emb(idx_hbm, table_hbm, out_hbm, idx_v, buf0, buf1, sem0, sem1):
        wid = lax.axis_index("s") * info.num_cores + lax.axis_index("c")
        base = wid * per_w
        pltpu.sync_copy(idx_hbm.at[pl.ds(base, per_w)], idx_v)
        bufs = (buf0, buf1)
        sems = (sem0, sem1)

        pltpu.async_copy(table_hbm.at[idx_v.at[pl.ds(0, G)]], buf0, sem0)

        def body(o, carry):
            for b in range(2):
                g = o * 2 + b
                nxt = g + 1

                @pl.when(nxt < n_g)
                def _():
                    pltpu.async_copy(
                        table_hbm.at[idx_v.at[pl.ds(nxt * G, G)]],
                        bufs[1 - b],
                        sems[1 - b],
                    )

                pltpu.make_async_copy(
                    table_hbm.at[idx_v.at[pl.ds(g * G, G)]], bufs[b], sems[b]
                ).wait()
                pltpu.sync_copy(bufs[b], out_hbm.at[pl.ds(base + g * G, G)])
            return carry

        lax.fori_loop(0, n_g // 2, body, 0)

    out = emb(idx, table)
    return out.reshape(B, N, L, D)


# speedup vs baseline: 1.0265x; 1.0259x over previous
import functools

import jax
import jax.numpy as jnp
from jax import lax
from jax.experimental import pallas as pl
from jax.experimental.pallas import tpu as pltpu
from jax.experimental.pallas import tpu_sc as plsc

G = 104


def kernel(input_feature, table):
    B, N, L = input_feature.shape
    V, D = table.shape
    T = B * N * L

    info = plsc.get_sparse_core_info()
    NW = info.num_cores * info.num_subcores
    per_w = T // NW
    n_g = per_w // G
    assert per_w * NW == T and n_g * G == per_w

    idx = input_feature.reshape(-1).astype(jnp.int32)
    mesh = plsc.VectorSubcoreMesh(core_axis_name="c", subcore_axis_name="s")

    g_per_b = (N * L) // G
    b_per_w = B // NW

    def emb(idx_hbm, table_hbm, out_hbm, idx_v, buf0, buf1, sem0, sem1):
        wid = lax.axis_index("s") * info.num_cores + lax.axis_index("c")
        base = wid * per_w
        b_base = wid * b_per_w
        pltpu.sync_copy(idx_hbm.at[pl.ds(base, per_w)], idx_v)
        bufs = (buf0, buf1)
        sems = (sem0, sem1)

        pltpu.async_copy(table_hbm.at[idx_v.at[pl.ds(0, G)]], buf0, sem0)

        def body(o, carry):
            for b in range(2):
                g = o * 2 + b
                nxt = g + 1

                @pl.when(nxt < n_g)
                def _():
                    pltpu.async_copy(
                        table_hbm.at[idx_v.at[pl.ds(nxt * G, G)]],
                        bufs[1 - b],
                        sems[1 - b],
                    )

                pltpu.make_async_copy(
                    table_hbm.at[idx_v.at[pl.ds(g * G, G)]], bufs[b], sems[b]
                ).wait()
                pltpu.sync_copy(
                    bufs[b],
                    out_hbm.at[b_base + g // g_per_b, pl.ds((g % g_per_b) * G, G)],
                )
            return carry

        lax.fori_loop(0, n_g // 2, body, 0)

    emb = functools.partial(
        pl.kernel,
        mesh=mesh,
        out_type=jax.ShapeDtypeStruct((B, N * L, D), jnp.float32),
        scratch_types=[
            pltpu.VMEM((per_w,), jnp.int32),
            pltpu.VMEM((G, D), jnp.float32),
            pltpu.VMEM((G, D), jnp.float32),
            pltpu.SemaphoreType.DMA,
            pltpu.SemaphoreType.DMA,
        ],
        compiler_params=pltpu.CompilerParams(use_tc_tiling_on_sc=False),
    )(emb)

    out = emb(idx, table)
    return out.reshape(B, N, L, D)
